# baseline (speedup 1.0000x reference)
import jax
import jax.numpy as jnp
from jax.experimental import pallas as pl
from jax.experimental.pallas import tpu as pltpu

_MASK_RATIO = 0.75


def _mae_body(x_ref, n_ref, we_ref, wdt_ref, be_ref, bd_ref, mt_ref,
              dec_ref, mask_ref, maug_ref, const_ref):
    LB, C, B = x_ref.shape
    L = n_ref.shape[0]
    KEEP = int(L * (1.0 - _MASK_RATIO))

    @pl.when((pl.program_id(0) + pl.program_id(1)) == 0)
    def _init():
        wdt = wdt_ref[...]
        bd_col = bd_ref[...]
        m_t = jnp.dot(wdt, jnp.transpose(we_ref[...]),
                      preferred_element_type=jnp.float32)
        c_col = jnp.dot(wdt, be_ref[...],
                        preferred_element_type=jnp.float32) + bd_col
        const_col = jnp.dot(wdt, mt_ref[...],
                            preferred_element_type=jnp.float32) + bd_col
        maug_ref[:, pl.ds(0, C)] = m_t
        maug_ref[:, pl.ds(C, 8)] = jnp.broadcast_to((c_col - const_col) * 0.125,
                                                    (C, 8))
        const_ref[...] = jnp.broadcast_to(const_col, (C, 8))

    nfull = n_ref[...]
    li = jax.lax.broadcasted_iota(jnp.int32, (L, 1), 0)
    base = pl.program_id(1) * LB
    maug = maug_ref[...]
    const_col = const_ref[:, 0:1]

    for j in range(LB):
        nl = n_ref[pl.ds(base + j, 1), :]
        pred = (nfull < nl) | ((nfull == nl) & (li < base + j))
        rank = jnp.sum(pred.astype(jnp.float32), axis=0, keepdims=True)
        keep = jnp.where(rank < KEEP, 1.0, 0.0)
        mask_ref[pl.ds(base + j, 1), :] = 1.0 - keep

        x_aug = jnp.concatenate(
            [x_ref[j] * keep, jnp.broadcast_to(keep, (8, B))], axis=0)
        dec = jnp.dot(maug, x_aug, preferred_element_type=jnp.float32)
        dec_ref[j] = dec + const_col


def kernel(x, noise, W_enc, b_enc, W_dec, b_dec, mask_token):
    B, C, H, W = x.shape
    L = H * W
    d_enc = W_enc.shape[1]
    x_t = x.transpose(2, 3, 1, 0).reshape(L, C, B)
    n_t = noise.T

    LB = 28
    BB = 128
    grid = (B // BB, L // LB)

    dec_t, mask_t = pl.pallas_call(
        _mae_body,
        grid=grid,
        in_specs=[
            pl.BlockSpec((LB, C, BB), lambda h, i: (i, 0, h)),
            pl.BlockSpec((L, BB), lambda h, i: (0, h)),
            pl.BlockSpec((C, d_enc), lambda h, i: (0, 0)),
            pl.BlockSpec((C, d_enc), lambda h, i: (0, 0)),
            pl.BlockSpec((d_enc, 1), lambda h, i: (0, 0)),
            pl.BlockSpec((C, 1), lambda h, i: (0, 0)),
            pl.BlockSpec((d_enc, 1), lambda h, i: (0, 0)),
        ],
        out_specs=[
            pl.BlockSpec((LB, C, BB), lambda h, i: (i, 0, h)),
            pl.BlockSpec((L, BB), lambda h, i: (0, h)),
        ],
        out_shape=[
            jax.ShapeDtypeStruct((L, C, B), jnp.float32),
            jax.ShapeDtypeStruct((L, B), jnp.float32),
        ],
        scratch_shapes=[
            pltpu.VMEM((C, C + 8), jnp.float32),
            pltpu.VMEM((C, 8), jnp.float32),
        ],
    )(x_t, n_t, W_enc, W_dec.T, b_enc.reshape(d_enc, 1),
      b_dec.reshape(C, 1), mask_token.reshape(d_enc, 1))
    return dec_t.transpose(2, 0, 1), mask_t.T


# speedup vs baseline: 1.0844x; 1.0844x over previous
import jax
import jax.numpy as jnp
from jax.experimental import pallas as pl
from jax.experimental.pallas import tpu as pltpu

_MASK_RATIO = 0.75


def _mae_body(x_ref, n_ref, we_ref, wdt_ref, be_ref, bd_ref, mt_ref,
              dec_ref, mask_ref, maug_ref, const_ref):
    LB, C, B = x_ref.shape
    L = n_ref.shape[0]
    KEEP = int(L * (1.0 - _MASK_RATIO))

    @pl.when((pl.program_id(0) + pl.program_id(1)) == 0)
    def _init():
        wdt = wdt_ref[...]
        bd_col = bd_ref[...]
        m_t = jnp.dot(wdt, jnp.transpose(we_ref[...]),
                      preferred_element_type=jnp.float32)
        c_col = jnp.dot(wdt, be_ref[...],
                        preferred_element_type=jnp.float32) + bd_col
        const_col = jnp.dot(wdt, mt_ref[...],
                            preferred_element_type=jnp.float32) + bd_col
        maug_ref[:, pl.ds(0, C)] = m_t
        maug_ref[:, pl.ds(C, 8)] = jnp.broadcast_to((c_col - const_col) * 0.125,
                                                    (C, 8))
        const_ref[...] = jnp.broadcast_to(const_col, (C, 8))

    nfull = n_ref[...]
    li = jax.lax.broadcasted_iota(jnp.int32, (L, 1), 0)
    base = pl.program_id(1) * LB
    maug = maug_ref[...]
    const_col = const_ref[:, 0:1]

    for j in range(LB):
        nl = n_ref[pl.ds(base + j, 1), :]
        pred = (nfull < nl) | ((nfull == nl) & (li < base + j))
        rank = jnp.sum(pred.astype(jnp.float32), axis=0, keepdims=True)
        keep = jnp.where(rank < KEEP, 1.0, 0.0)
        mask_ref[pl.ds(base + j, 1), :] = 1.0 - keep

        x_aug = jnp.concatenate(
            [x_ref[j] * keep, jnp.broadcast_to(keep, (8, B))], axis=0)
        dec = jnp.dot(maug, x_aug, preferred_element_type=jnp.float32)
        dec_ref[j] = dec + const_col


def kernel(x, noise, W_enc, b_enc, W_dec, b_dec, mask_token):
    B, C, H, W = x.shape
    L = H * W
    d_enc = W_enc.shape[1]
    x_t = x.transpose(2, 3, 1, 0).reshape(L, C, B)
    n_t = noise.T

    LB = 98
    BB = 128
    grid = (B // BB, L // LB)

    dec_t, mask_t = pl.pallas_call(
        _mae_body,
        grid=grid,
        in_specs=[
            pl.BlockSpec((LB, C, BB), lambda h, i: (i, 0, h)),
            pl.BlockSpec((L, BB), lambda h, i: (0, h)),
            pl.BlockSpec((C, d_enc), lambda h, i: (0, 0)),
            pl.BlockSpec((C, d_enc), lambda h, i: (0, 0)),
            pl.BlockSpec((d_enc, 1), lambda h, i: (0, 0)),
            pl.BlockSpec((C, 1), lambda h, i: (0, 0)),
            pl.BlockSpec((d_enc, 1), lambda h, i: (0, 0)),
        ],
        out_specs=[
            pl.BlockSpec((LB, C, BB), lambda h, i: (i, 0, h)),
            pl.BlockSpec((L, BB), lambda h, i: (0, h)),
        ],
        out_shape=[
            jax.ShapeDtypeStruct((L, C, B), jnp.float32),
            jax.ShapeDtypeStruct((L, B), jnp.float32),
        ],
        scratch_shapes=[
            pltpu.VMEM((C, C + 8), jnp.float32),
            pltpu.VMEM((C, 8), jnp.float32),
        ],
    )(x_t, n_t, W_enc, W_dec.T, b_enc.reshape(d_enc, 1),
      b_dec.reshape(C, 1), mask_token.reshape(d_enc, 1))
    return dec_t.transpose(2, 0, 1), mask_t.T
